# baseline (speedup 1.0000x reference)
import functools

import jax
import jax.numpy as jnp
from jax import lax
from jax.experimental import pallas as pl
from jax.experimental.pallas import tpu as pltpu
from jax.experimental.pallas import tpu_sc as plsc

NC = 2
NS = 16
NW = NC * NS
LANES = 16

N = 10000
E = 320000
D = 128
OUT = 64

C = 128
KC = -(-E // (NW * C))
KC = ((KC + 3) // 4) * 4
EPW = KC * C
EP = EPW * NW
N_PAD = ((N + 1 + 255) // 256) * 256
ROWS_PER = N_PAD // NS
ZROWS = ((ROWS_PER + LANES - 1) // LANES) * LANES


def _mesh():
    return plsc.VectorSubcoreMesh(
        core_axis_name="c", subcore_axis_name="s",
        num_cores=NC, num_subcores=NS)


@functools.cache
def _sc_degrees_call():
    return functools.partial(
        pl.kernel,
        out_type=[
            jax.ShapeDtypeStruct((NC * N_PAD,), jnp.float32),
            jax.ShapeDtypeStruct((NC * N_PAD,), jnp.float32),
        ],
        mesh=_mesh(),
        scratch_types=[
            pltpu.VMEM((KC, C), jnp.int32),
            pltpu.VMEM((KC, C), jnp.int32),
            pltpu.VMEM((C,), jnp.float32),
            pltpu.VMEM((ZROWS,), jnp.float32),
            pltpu.VMEM_SHARED((N_PAD,), jnp.float32),
            pltpu.VMEM_SHARED((N_PAD,), jnp.float32),
        ],
    )(_sc_degrees_body)


def _sc_degrees_body(src_hbm, dst_hbm, outs, outd, src_v, dst_v, ones_v,
                     zero_v, accs, accd):
    c = lax.axis_index("c")
    s = lax.axis_index("s")
    wid = s * NC + c

    for i in range(C // LANES):
        ones_v[pl.ds(i * LANES, LANES)] = jnp.ones((LANES,), jnp.float32)
    nz = ZROWS // LANES
    for i in range(nz):
        zero_v[pl.ds(i * LANES, LANES)] = jnp.zeros((LANES,), jnp.float32)
    base = s * ROWS_PER
    pltpu.sync_copy(zero_v.at[pl.ds(0, ROWS_PER)], accs.at[pl.ds(base, ROWS_PER)])
    pltpu.sync_copy(zero_v.at[pl.ds(0, ROWS_PER)], accd.at[pl.ds(base, ROWS_PER)])

    pltpu.sync_copy(src_hbm.at[wid], src_v)
    pltpu.sync_copy(dst_hbm.at[wid], dst_v)
    plsc.subcore_barrier()

    @pl.loop(0, KC)
    def _(j):
        pltpu.sync_copy(ones_v, accs.at[src_v.at[j]], add=True)
        pltpu.sync_copy(ones_v, accd.at[dst_v.at[j]], add=True)

    plsc.subcore_barrier()
    obase = c * N_PAD + base
    pltpu.sync_copy(accs.at[pl.ds(base, ROWS_PER)], zero_v.at[pl.ds(0, ROWS_PER)])
    pltpu.sync_copy(zero_v.at[pl.ds(0, ROWS_PER)], outs.at[pl.ds(obase, ROWS_PER)])
    pltpu.sync_copy(accd.at[pl.ds(base, ROWS_PER)], zero_v.at[pl.ds(0, ROWS_PER)])
    pltpu.sync_copy(zero_v.at[pl.ds(0, ROWS_PER)], outd.at[pl.ds(obase, ROWS_PER)])


@functools.cache
def _sc_segsum_call():
    return functools.partial(
        pl.kernel,
        out_type=jax.ShapeDtypeStruct((NC, N_PAD, D), jnp.float32),
        mesh=_mesh(),
        scratch_types=[
            pltpu.VMEM((KC, C), jnp.int32),
        ] + [pltpu.VMEM((C,), jnp.int32) for _ in range(4)] + [
            pltpu.VMEM((C, D), jnp.float32),
            pltpu.VMEM((C, D), jnp.float32),
            pltpu.VMEM_SHARED((N_PAD, D), jnp.float32),
        ] + [pltpu.SemaphoreType.DMA] * 6,
    )(_sc_segsum_body)


def _sc_segsum_body(x_hbm, src_hbm, dst_hbm, out, dst_v, i0, i1, i2, i3,
                    r0, r1, acc, *sems):
    c = lax.axis_index("c")
    s = lax.axis_index("s")
    wid = s * NC + c
    idx = (i0, i1, i2, i3)
    rows = (r0, r1)
    isems = sems[0:4]
    rsems = sems[4:6]

    @pl.loop(0, C)
    def _(r):
        for i in range(D // LANES):
            r0[r, pl.ds(i * LANES, LANES)] = jnp.zeros((LANES,), jnp.float32)
    base = s * ROWS_PER
    nfull = ROWS_PER // C
    for t in range(nfull):
        pltpu.sync_copy(r0, acc.at[pl.ds(base + t * C, C)])
    rem = ROWS_PER - nfull * C
    if rem:
        pltpu.sync_copy(r0.at[pl.ds(0, rem)],
                        acc.at[pl.ds(base + nfull * C, rem)])

    pltpu.sync_copy(dst_hbm.at[wid], dst_v)
    plsc.subcore_barrier()

    for k in range(4):
        pltpu.async_copy(src_hbm.at[wid, k], idx[k], isems[k])
    for k in range(2):
        pltpu.make_async_copy(src_hbm.at[wid, k], idx[k], isems[k]).wait()
        pltpu.async_copy(x_hbm.at[idx[k]], rows[k], rsems[k])

    @pl.loop(0, KC, step=4)
    def _(j0):
        for b in range(4):
            j = j0 + b
            pltpu.make_async_copy(
                x_hbm.at[idx[b]], rows[b % 2], rsems[b % 2]).wait()
            pltpu.sync_copy(rows[b % 2], acc.at[dst_v.at[j]], add=True)
            @pl.when(j + 2 < KC)
            def _():
                pltpu.make_async_copy(
                    src_hbm.at[wid, j + 2], idx[(b + 2) % 4],
                    isems[(b + 2) % 4]).wait()
                pltpu.async_copy(x_hbm.at[idx[(b + 2) % 4]], rows[b % 2],
                                 rsems[b % 2])
            @pl.when(j + 4 < KC)
            def _():
                pltpu.async_copy(src_hbm.at[wid, j + 4], idx[b], isems[b])

    plsc.subcore_barrier()
    pltpu.sync_copy(acc.at[pl.ds(base, ROWS_PER)],
                    out.at[c, pl.ds(base, ROWS_PER)])


def _rs(deg_ref):
    d = deg_ref[0] + deg_ref[1]
    return lax.rsqrt(jnp.maximum(d, 1.0))[:N]


def _tc_mm0(h_ref, w0_ref, out_ref):
    out_ref[...] = jnp.dot(h_ref[...], w0_ref[...],
                           preferred_element_type=jnp.float32)


def _tc_scale0(p_ref, degs_ref, p0_ref):
    p0_ref[:N, :] = p_ref[...] * _rs(degs_ref)
    p0_ref[N:, :] = jnp.zeros((N_PAD - N, D), jnp.float32)


def _bn_relu(x, gamma, beta):
    mu = jnp.mean(x, axis=0, keepdims=True)
    var = jnp.mean((x - mu) ** 2, axis=0, keepdims=True)
    return jnp.maximum(gamma * (x - mu) * lax.rsqrt(var + 1e-3) + beta, 0.0)


def _tc_mid(q0_ref, degd_ref, degs_ref, b0_ref, g0_ref, be0_ref, w1_ref,
            h1_ref, p1_ref):
    rs_in = _rs(degd_ref)
    q0 = q0_ref[0, :N, :] + q0_ref[1, :N, :]
    x = q0 * rs_in + b0_ref[...]
    h1 = _bn_relu(x, g0_ref[...], be0_ref[...])
    h1_ref[...] = h1
    rs_out = _rs(degs_ref)
    p1 = jnp.dot(h1, w1_ref[...], preferred_element_type=jnp.float32)
    p1_ref[:N, :] = p1 * rs_out
    p1_ref[N:, :] = jnp.zeros((N_PAD - N, D), jnp.float32)


def _tc_final(q1_ref, degd_ref, b1_ref, g1_ref, be1_ref, h1_ref, wp_ref,
              bp_ref, out_ref):
    rs_in = _rs(degd_ref)
    q1 = q1_ref[0, :N, :] + q1_ref[1, :N, :]
    x = q1 * rs_in + b1_ref[...]
    h2 = _bn_relu(x, g1_ref[...], be1_ref[...]) + h1_ref[...]
    pooled = jnp.mean(h2, axis=0, keepdims=True)
    out_ref[...] = (
        jnp.dot(pooled, wp_ref[...], preferred_element_type=jnp.float32)
        + bp_ref[...])


_mm0 = pl.pallas_call(
    _tc_mm0,
    out_shape=jax.ShapeDtypeStruct((N, D), jnp.float32))
_scale0 = pl.pallas_call(
    _tc_scale0,
    out_shape=jax.ShapeDtypeStruct((N_PAD, D), jnp.float32))
_mid = pl.pallas_call(
    _tc_mid,
    out_shape=[jax.ShapeDtypeStruct((N, D), jnp.float32),
               jax.ShapeDtypeStruct((N_PAD, D), jnp.float32)])
_final = pl.pallas_call(
    _tc_final,
    out_shape=jax.ShapeDtypeStruct((1, OUT), jnp.float32))


def kernel(h, edge_index, W0, b0, gamma0, beta0, W1, b1, gamma1, beta1, Wp, bp):
    src = edge_index[0].astype(jnp.int32)
    dst = edge_index[1].astype(jnp.int32)
    padr = N + (jnp.arange(EP - E, dtype=jnp.int32) % (N_PAD - N))
    srcp = jnp.concatenate([src, padr]).reshape(NW, KC, C)
    dstp = jnp.concatenate([dst, padr]).reshape(NW, KC, C)

    degs_p, degd_p = _sc_degrees_call()(srcp, dstp)
    degs_c = degs_p.reshape(NC, N_PAD, 1)
    degd_c = degd_p.reshape(NC, N_PAD, 1)

    b0r = b0.reshape(1, D)
    g0r = gamma0.reshape(1, D)
    be0r = beta0.reshape(1, D)
    b1r = b1.reshape(1, D)
    g1r = gamma1.reshape(1, D)
    be1r = beta1.reshape(1, D)
    bpr = bp.reshape(1, OUT)

    p0 = _scale0(_mm0(h, W0), degs_c)
    segsum = _sc_segsum_call()
    q0p = segsum(p0, srcp, dstp)
    h1, p1 = _mid(q0p, degd_c, degs_c, b0r, g0r, be0r, W1)
    q1p = segsum(p1, srcp, dstp)
    return _final(q1p, degd_c, b1r, g1r, be1r, h1, Wp, bpr)


# speedup vs baseline: 1.0380x; 1.0380x over previous
import functools

import jax
import jax.numpy as jnp
from jax import lax
from jax.experimental import pallas as pl
from jax.experimental.pallas import tpu as pltpu
from jax.experimental.pallas import tpu_sc as plsc

NC = 2
NS = 16
NW = NC * NS
LANES = 16

N = 10000
E = 320000
D = 128
OUT = 64

C = 120
KC = -(-E // (NW * C))
KC = ((KC + 2) // 3) * 3
EPW = KC * C
EP = EPW * NW
N_PAD = ((N + 1 + 255) // 256) * 256
ROWS_PER = N_PAD // NS
ZROWS = ((ROWS_PER + LANES - 1) // LANES) * LANES


def _mesh():
    return plsc.VectorSubcoreMesh(
        core_axis_name="c", subcore_axis_name="s",
        num_cores=NC, num_subcores=NS)


@functools.cache
def _sc_degrees_call():
    return functools.partial(
        pl.kernel,
        out_type=[
            jax.ShapeDtypeStruct((NC * N_PAD,), jnp.float32),
            jax.ShapeDtypeStruct((NC * N_PAD,), jnp.float32),
        ],
        mesh=_mesh(),
        scratch_types=[
            pltpu.VMEM((KC, C), jnp.int32),
            pltpu.VMEM((KC, C), jnp.int32),
            pltpu.VMEM((C,), jnp.float32),
            pltpu.VMEM((ZROWS,), jnp.float32),
            pltpu.VMEM_SHARED((N_PAD,), jnp.float32),
            pltpu.VMEM_SHARED((N_PAD,), jnp.float32),
        ],
    )(_sc_degrees_body)


def _sc_degrees_body(src_hbm, dst_hbm, outs, outd, src_v, dst_v, ones_v,
                     zero_v, accs, accd):
    c = lax.axis_index("c")
    s = lax.axis_index("s")
    wid = s * NC + c

    for i in range(C // LANES):
        ones_v[pl.ds(i * LANES, LANES)] = jnp.ones((LANES,), jnp.float32)
    if C % LANES:
        ones_v[pl.ds(C - LANES, LANES)] = jnp.ones((LANES,), jnp.float32)
    nz = ZROWS // LANES
    for i in range(nz):
        zero_v[pl.ds(i * LANES, LANES)] = jnp.zeros((LANES,), jnp.float32)
    base = s * ROWS_PER
    pltpu.sync_copy(zero_v.at[pl.ds(0, ROWS_PER)], accs.at[pl.ds(base, ROWS_PER)])
    pltpu.sync_copy(zero_v.at[pl.ds(0, ROWS_PER)], accd.at[pl.ds(base, ROWS_PER)])

    pltpu.sync_copy(src_hbm.at[wid], src_v)
    pltpu.sync_copy(dst_hbm.at[wid], dst_v)
    plsc.subcore_barrier()

    @pl.loop(0, KC)
    def _(j):
        pltpu.sync_copy(ones_v, accs.at[src_v.at[j]], add=True)
        pltpu.sync_copy(ones_v, accd.at[dst_v.at[j]], add=True)

    plsc.subcore_barrier()
    obase = c * N_PAD + base
    pltpu.sync_copy(accs.at[pl.ds(base, ROWS_PER)], zero_v.at[pl.ds(0, ROWS_PER)])
    pltpu.sync_copy(zero_v.at[pl.ds(0, ROWS_PER)], outs.at[pl.ds(obase, ROWS_PER)])
    pltpu.sync_copy(accd.at[pl.ds(base, ROWS_PER)], zero_v.at[pl.ds(0, ROWS_PER)])
    pltpu.sync_copy(zero_v.at[pl.ds(0, ROWS_PER)], outd.at[pl.ds(obase, ROWS_PER)])


@functools.cache
def _sc_segsum_call():
    return functools.partial(
        pl.kernel,
        out_type=jax.ShapeDtypeStruct((NC, N_PAD, D), jnp.float32),
        mesh=_mesh(),
        scratch_types=[
            pltpu.VMEM((C,), jnp.int32) for _ in range(3)
        ] + [
            pltpu.VMEM((C,), jnp.int32) for _ in range(3)
        ] + [
            pltpu.VMEM((C, D), jnp.float32) for _ in range(3)
        ] + [
            pltpu.VMEM_SHARED((N_PAD, D), jnp.float32),
        ] + [pltpu.SemaphoreType.DMA] * 12,
    )(_sc_segsum_body)


def _sc_segsum_body(x_hbm, src_hbm, dst_hbm, out, s0, s1, s2, d0, d1, d2,
                    r0, r1, r2, acc, *sems):
    c = lax.axis_index("c")
    s = lax.axis_index("s")
    wid = s * NC + c
    sidx = (s0, s1, s2)
    didx = (d0, d1, d2)
    rows = (r0, r1, r2)
    isems = sems[0:3]
    dsems = sems[3:6]
    rsems = sems[6:9]
    ssems = sems[9:12]

    @pl.loop(0, C)
    def _(r):
        for i in range(D // LANES):
            r0[r, pl.ds(i * LANES, LANES)] = jnp.zeros((LANES,), jnp.float32)
    base = s * ROWS_PER
    nfull = ROWS_PER // C
    for t in range(nfull):
        pltpu.sync_copy(r0, acc.at[pl.ds(base + t * C, C)])
    rem = ROWS_PER - nfull * C
    if rem:
        pltpu.sync_copy(r0.at[pl.ds(0, rem)],
                        acc.at[pl.ds(base + nfull * C, rem)])
    plsc.subcore_barrier()

    for k in range(3):
        pltpu.async_copy(src_hbm.at[wid, k], sidx[k], isems[k])
    for k in range(2):
        pltpu.async_copy(dst_hbm.at[wid, k], didx[k], dsems[k])
    for k in range(2):
        pltpu.make_async_copy(src_hbm.at[wid, k], sidx[k], isems[k]).wait()
        pltpu.async_copy(x_hbm.at[sidx[k]], rows[k], rsems[k])

    @pl.loop(0, KC, step=3)
    def _(j0):
        for b in range(3):
            j = j0 + b
            pltpu.make_async_copy(
                x_hbm.at[sidx[b]], rows[b], rsems[b]).wait()
            @pl.when(j + 3 < KC)
            def _():
                pltpu.async_copy(src_hbm.at[wid, j + 3], sidx[b], isems[b])
            pltpu.make_async_copy(
                dst_hbm.at[wid, j], didx[b], dsems[b]).wait()
            pltpu.async_copy(rows[b], acc.at[didx[b]], ssems[b],
                             add=True)

            def _drain(bb=b):
                pltpu.make_async_copy(
                    rows[(bb + 2) % 3], acc.at[didx[(bb + 2) % 3]],
                    ssems[(bb + 2) % 3]).wait()
            if b == 0:
                @pl.when(j0 > 0)
                def _():
                    _drain()
            else:
                _drain()

            @pl.when(j + 2 < KC)
            def _():
                pltpu.async_copy(dst_hbm.at[wid, j + 2], didx[(b + 2) % 3],
                                 dsems[(b + 2) % 3])
                pltpu.make_async_copy(
                    src_hbm.at[wid, j + 2], sidx[(b + 2) % 3],
                    isems[(b + 2) % 3]).wait()
                pltpu.async_copy(x_hbm.at[sidx[(b + 2) % 3]],
                                 rows[(b + 2) % 3], rsems[(b + 2) % 3])

    pltpu.make_async_copy(
        rows[(KC - 1) % 3], acc.at[didx[(KC - 1) % 3]],
        ssems[(KC - 1) % 3]).wait()
    plsc.subcore_barrier()
    pltpu.sync_copy(acc.at[pl.ds(base, ROWS_PER)],
                    out.at[c, pl.ds(base, ROWS_PER)])


def _rs(deg_ref):
    d = deg_ref[0] + deg_ref[1]
    return lax.rsqrt(jnp.maximum(d, 1.0))[:N]


def _tc_mm0(h_ref, w0_ref, out_ref):
    out_ref[...] = jnp.dot(h_ref[...], w0_ref[...],
                           preferred_element_type=jnp.float32)


def _tc_scale0(p_ref, degs_ref, p0_ref):
    p0_ref[:N, :] = p_ref[...] * _rs(degs_ref)
    p0_ref[N:, :] = jnp.zeros((N_PAD - N, D), jnp.float32)


def _bn_relu(x, gamma, beta):
    mu = jnp.mean(x, axis=0, keepdims=True)
    var = jnp.mean((x - mu) ** 2, axis=0, keepdims=True)
    return jnp.maximum(gamma * (x - mu) * lax.rsqrt(var + 1e-3) + beta, 0.0)


def _tc_mid(q0_ref, degd_ref, degs_ref, b0_ref, g0_ref, be0_ref, w1_ref,
            h1_ref, p1_ref):
    rs_in = _rs(degd_ref)
    q0 = q0_ref[0, :N, :] + q0_ref[1, :N, :]
    x = q0 * rs_in + b0_ref[...]
    h1 = _bn_relu(x, g0_ref[...], be0_ref[...])
    h1_ref[...] = h1
    rs_out = _rs(degs_ref)
    p1 = jnp.dot(h1, w1_ref[...], preferred_element_type=jnp.float32)
    p1_ref[:N, :] = p1 * rs_out
    p1_ref[N:, :] = jnp.zeros((N_PAD - N, D), jnp.float32)


def _tc_final(q1_ref, degd_ref, b1_ref, g1_ref, be1_ref, h1_ref, wp_ref,
              bp_ref, out_ref):
    rs_in = _rs(degd_ref)
    q1 = q1_ref[0, :N, :] + q1_ref[1, :N, :]
    x = q1 * rs_in + b1_ref[...]
    h2 = _bn_relu(x, g1_ref[...], be1_ref[...]) + h1_ref[...]
    pooled = jnp.mean(h2, axis=0, keepdims=True)
    out_ref[...] = (
        jnp.dot(pooled, wp_ref[...], preferred_element_type=jnp.float32)
        + bp_ref[...])


_mm0 = pl.pallas_call(
    _tc_mm0,
    out_shape=jax.ShapeDtypeStruct((N, D), jnp.float32))
_scale0 = pl.pallas_call(
    _tc_scale0,
    out_shape=jax.ShapeDtypeStruct((N_PAD, D), jnp.float32))
_mid = pl.pallas_call(
    _tc_mid,
    out_shape=[jax.ShapeDtypeStruct((N, D), jnp.float32),
               jax.ShapeDtypeStruct((N_PAD, D), jnp.float32)])
_final = pl.pallas_call(
    _tc_final,
    out_shape=jax.ShapeDtypeStruct((1, OUT), jnp.float32))


def kernel(h, edge_index, W0, b0, gamma0, beta0, W1, b1, gamma1, beta1, Wp, bp):
    src = edge_index[0].astype(jnp.int32)
    dst = edge_index[1].astype(jnp.int32)
    padr = N + (jnp.arange(EP - E, dtype=jnp.int32) % (N_PAD - N))
    srcp = jnp.concatenate([src, padr]).reshape(NW, KC, C)
    dstp = jnp.concatenate([dst, padr]).reshape(NW, KC, C)

    degs_p, degd_p = _sc_degrees_call()(srcp, dstp)
    degs_c = degs_p.reshape(NC, N_PAD, 1)
    degd_c = degd_p.reshape(NC, N_PAD, 1)

    b0r = b0.reshape(1, D)
    g0r = gamma0.reshape(1, D)
    be0r = beta0.reshape(1, D)
    b1r = b1.reshape(1, D)
    g1r = gamma1.reshape(1, D)
    be1r = beta1.reshape(1, D)
    bpr = bp.reshape(1, OUT)

    p0 = _scale0(_mm0(h, W0), degs_c)
    segsum = _sc_segsum_call()
    q0p = segsum(p0, srcp, dstp)
    h1, p1 = _mid(q0p, degd_c, degs_c, b0r, g0r, be0r, W1)
    q1p = segsum(p1, srcp, dstp)
    return _final(q1p, degd_c, b1r, g1r, be1r, h1, Wp, bpr)


# speedup vs baseline: 1.0887x; 1.0488x over previous
import functools

import jax
import jax.numpy as jnp
from jax import lax
from jax.experimental import pallas as pl
from jax.experimental.pallas import tpu as pltpu
from jax.experimental.pallas import tpu_sc as plsc

NC = 2
NS = 16
NW = NC * NS
LANES = 16

N = 10000
E = 320000
D = 128
OUT = 64

C = 120
KC = -(-E // (NW * C))
KC = ((KC + 2) // 3) * 3
EPW = KC * C
EP = EPW * NW
N_PAD = ((N + 1 + 255) // 256) * 256
ROWS_PER = N_PAD // NS
ZROWS = ((ROWS_PER + LANES - 1) // LANES) * LANES


def _mesh():
    return plsc.VectorSubcoreMesh(
        core_axis_name="c", subcore_axis_name="s",
        num_cores=NC, num_subcores=NS)


@functools.cache
def _sc_degrees_call():
    return functools.partial(
        pl.kernel,
        out_type=[
            jax.ShapeDtypeStruct((NC * N_PAD,), jnp.float32),
            jax.ShapeDtypeStruct((NC * N_PAD,), jnp.float32),
        ],
        mesh=_mesh(),
        scratch_types=[
            pltpu.VMEM((KC, C), jnp.int32),
            pltpu.VMEM((KC, C), jnp.int32),
            pltpu.VMEM((C,), jnp.float32),
            pltpu.VMEM((ZROWS,), jnp.float32),
            pltpu.VMEM_SHARED((N_PAD,), jnp.float32),
            pltpu.VMEM_SHARED((N_PAD,), jnp.float32),
        ],
    )(_sc_degrees_body)


def _sc_degrees_body(src_hbm, dst_hbm, outs, outd, src_v, dst_v, ones_v,
                     zero_v, accs, accd):
    c = lax.axis_index("c")
    s = lax.axis_index("s")
    wid = s * NC + c

    for i in range(C // LANES):
        ones_v[pl.ds(i * LANES, LANES)] = jnp.ones((LANES,), jnp.float32)
    if C % LANES:
        ones_v[pl.ds(C - LANES, LANES)] = jnp.ones((LANES,), jnp.float32)
    nz = ZROWS // LANES
    for i in range(nz):
        zero_v[pl.ds(i * LANES, LANES)] = jnp.zeros((LANES,), jnp.float32)
    base = s * ROWS_PER
    pltpu.sync_copy(zero_v.at[pl.ds(0, ROWS_PER)], accs.at[pl.ds(base, ROWS_PER)])
    pltpu.sync_copy(zero_v.at[pl.ds(0, ROWS_PER)], accd.at[pl.ds(base, ROWS_PER)])

    pltpu.sync_copy(src_hbm.at[wid], src_v)
    pltpu.sync_copy(dst_hbm.at[wid], dst_v)
    plsc.subcore_barrier()

    @pl.loop(0, KC)
    def _(j):
        pltpu.sync_copy(ones_v, accs.at[src_v.at[j]], add=True)
        pltpu.sync_copy(ones_v, accd.at[dst_v.at[j]], add=True)

    plsc.subcore_barrier()
    obase = c * N_PAD + base
    pltpu.sync_copy(accs.at[pl.ds(base, ROWS_PER)], zero_v.at[pl.ds(0, ROWS_PER)])
    pltpu.sync_copy(zero_v.at[pl.ds(0, ROWS_PER)], outs.at[pl.ds(obase, ROWS_PER)])
    pltpu.sync_copy(accd.at[pl.ds(base, ROWS_PER)], zero_v.at[pl.ds(0, ROWS_PER)])
    pltpu.sync_copy(zero_v.at[pl.ds(0, ROWS_PER)], outd.at[pl.ds(obase, ROWS_PER)])


@functools.cache
def _sc_segsum_call():
    return functools.partial(
        pl.kernel,
        out_type=jax.ShapeDtypeStruct((NC, N_PAD, D), jnp.float32),
        mesh=_mesh(),
        scratch_types=[
            pltpu.VMEM((C,), jnp.int32) for _ in range(3)
        ] + [
            pltpu.VMEM((C,), jnp.int32) for _ in range(3)
        ] + [
            pltpu.VMEM((C, D), jnp.float32) for _ in range(3)
        ] + [
            pltpu.VMEM_SHARED((N_PAD, D), jnp.float32),
        ] + [pltpu.SemaphoreType.DMA] * 12,
    )(_sc_segsum_body)


def _sc_segsum_body(x_hbm, src_hbm, dst_hbm, out, s0, s1, s2, d0, d1, d2,
                    r0, r1, r2, acc, *sems):
    c = lax.axis_index("c")
    s = lax.axis_index("s")
    wid = s * NC + c
    sidx = (s0, s1, s2)
    didx = (d0, d1, d2)
    rows = (r0, r1, r2)
    isems = sems[0:3]
    dsems = sems[3:6]
    rsems = sems[6:9]
    ssems = sems[9:12]

    @pl.loop(0, C)
    def _(r):
        for i in range(D // LANES):
            r0[r, pl.ds(i * LANES, LANES)] = jnp.zeros((LANES,), jnp.float32)
    base = s * ROWS_PER
    nfull = ROWS_PER // C
    for t in range(nfull):
        pltpu.sync_copy(r0, acc.at[pl.ds(base + t * C, C)])
    rem = ROWS_PER - nfull * C
    if rem:
        pltpu.sync_copy(r0.at[pl.ds(0, rem)],
                        acc.at[pl.ds(base + nfull * C, rem)])
    plsc.subcore_barrier()

    for k in range(3):
        pltpu.async_copy(src_hbm.at[wid, k], sidx[k], isems[k])
    for k in range(2):
        pltpu.async_copy(dst_hbm.at[wid, k], didx[k], dsems[k])
    for k in range(2):
        pltpu.make_async_copy(src_hbm.at[wid, k], sidx[k], isems[k]).wait()
        pltpu.async_copy(x_hbm.at[sidx[k]], rows[k], rsems[k])

    @pl.loop(0, KC, step=3)
    def _(j0):
        for b in range(3):
            j = j0 + b
            pltpu.make_async_copy(
                x_hbm.at[sidx[b]], rows[b], rsems[b]).wait()
            @pl.when(j + 3 < KC)
            def _():
                pltpu.async_copy(src_hbm.at[wid, j + 3], sidx[b], isems[b])
            def _drain(bb=b):
                pltpu.make_async_copy(
                    rows[(bb + 2) % 3], acc.at[didx[(bb + 2) % 3]],
                    ssems[(bb + 2) % 3]).wait()
            if b == 0:
                @pl.when(j0 > 0)
                def _():
                    _drain()
            else:
                _drain()

            @pl.when(j + 2 < KC)
            def _():
                pltpu.make_async_copy(
                    src_hbm.at[wid, j + 2], sidx[(b + 2) % 3],
                    isems[(b + 2) % 3]).wait()
                pltpu.async_copy(x_hbm.at[sidx[(b + 2) % 3]],
                                 rows[(b + 2) % 3], rsems[(b + 2) % 3])
                pltpu.async_copy(dst_hbm.at[wid, j + 2], didx[(b + 2) % 3],
                                 dsems[(b + 2) % 3])

            pltpu.make_async_copy(
                dst_hbm.at[wid, j], didx[b], dsems[b]).wait()
            pltpu.async_copy(rows[b], acc.at[didx[b]], ssems[b],
                             add=True)

    pltpu.make_async_copy(
        rows[(KC - 1) % 3], acc.at[didx[(KC - 1) % 3]],
        ssems[(KC - 1) % 3]).wait()
    plsc.subcore_barrier()
    pltpu.sync_copy(acc.at[pl.ds(base, ROWS_PER)],
                    out.at[c, pl.ds(base, ROWS_PER)])


def _rs(deg_ref):
    d = deg_ref[0] + deg_ref[1]
    return lax.rsqrt(jnp.maximum(d, 1.0))[:N]


def _tc_mm0(h_ref, w0_ref, out_ref):
    out_ref[...] = jnp.dot(h_ref[...], w0_ref[...],
                           preferred_element_type=jnp.float32)


def _tc_scale0(p_ref, degs_ref, p0_ref):
    p0_ref[:N, :] = p_ref[...] * _rs(degs_ref)
    p0_ref[N:, :] = jnp.zeros((N_PAD - N, D), jnp.float32)


def _bn_relu(x, gamma, beta):
    mu = jnp.mean(x, axis=0, keepdims=True)
    var = jnp.mean((x - mu) ** 2, axis=0, keepdims=True)
    return jnp.maximum(gamma * (x - mu) * lax.rsqrt(var + 1e-3) + beta, 0.0)


def _tc_mid(q0_ref, degd_ref, degs_ref, b0_ref, g0_ref, be0_ref, w1_ref,
            h1_ref, p1_ref):
    rs_in = _rs(degd_ref)
    q0 = q0_ref[0, :N, :] + q0_ref[1, :N, :]
    x = q0 * rs_in + b0_ref[...]
    h1 = _bn_relu(x, g0_ref[...], be0_ref[...])
    h1_ref[...] = h1
    rs_out = _rs(degs_ref)
    p1 = jnp.dot(h1, w1_ref[...], preferred_element_type=jnp.float32)
    p1_ref[:N, :] = p1 * rs_out
    p1_ref[N:, :] = jnp.zeros((N_PAD - N, D), jnp.float32)


def _tc_final(q1_ref, degd_ref, b1_ref, g1_ref, be1_ref, h1_ref, wp_ref,
              bp_ref, out_ref):
    rs_in = _rs(degd_ref)
    q1 = q1_ref[0, :N, :] + q1_ref[1, :N, :]
    x = q1 * rs_in + b1_ref[...]
    h2 = _bn_relu(x, g1_ref[...], be1_ref[...]) + h1_ref[...]
    pooled = jnp.mean(h2, axis=0, keepdims=True)
    out_ref[...] = (
        jnp.dot(pooled, wp_ref[...], preferred_element_type=jnp.float32)
        + bp_ref[...])


_mm0 = pl.pallas_call(
    _tc_mm0,
    out_shape=jax.ShapeDtypeStruct((N, D), jnp.float32))
_scale0 = pl.pallas_call(
    _tc_scale0,
    out_shape=jax.ShapeDtypeStruct((N_PAD, D), jnp.float32))
_mid = pl.pallas_call(
    _tc_mid,
    out_shape=[jax.ShapeDtypeStruct((N, D), jnp.float32),
               jax.ShapeDtypeStruct((N_PAD, D), jnp.float32)])
_final = pl.pallas_call(
    _tc_final,
    out_shape=jax.ShapeDtypeStruct((1, OUT), jnp.float32))


def kernel(h, edge_index, W0, b0, gamma0, beta0, W1, b1, gamma1, beta1, Wp, bp):
    src = edge_index[0].astype(jnp.int32)
    dst = edge_index[1].astype(jnp.int32)
    padr = N + (jnp.arange(EP - E, dtype=jnp.int32) % (N_PAD - N))
    srcp = jnp.concatenate([src, padr]).reshape(NW, KC, C)
    dstp = jnp.concatenate([dst, padr]).reshape(NW, KC, C)

    degs_p, degd_p = _sc_degrees_call()(srcp, dstp)
    degs_c = degs_p.reshape(NC, N_PAD, 1)
    degd_c = degd_p.reshape(NC, N_PAD, 1)

    b0r = b0.reshape(1, D)
    g0r = gamma0.reshape(1, D)
    be0r = beta0.reshape(1, D)
    b1r = b1.reshape(1, D)
    g1r = gamma1.reshape(1, D)
    be1r = beta1.reshape(1, D)
    bpr = bp.reshape(1, OUT)

    p0 = _scale0(_mm0(h, W0), degs_c)
    segsum = _sc_segsum_call()
    q0p = segsum(p0, srcp, dstp)
    h1, p1 = _mid(q0p, degd_c, degs_c, b0r, g0r, be0r, W1)
    q1p = segsum(p1, srcp, dstp)
    return _final(q1p, degd_c, b1r, g1r, be1r, h1, Wp, bpr)
